# Initial kernel scaffold; baseline (speedup 1.0000x reference)
import functools

import jax
import jax.numpy as jnp
from jax import lax
from jax.experimental import pallas as pl
from jax.experimental.pallas import tpu as pltpu
from jax.experimental.pallas import tpu_sc as plsc

N = 10000
IN_DIM = 128
HID = 64
OUT_DIM = 32
NEG = 0.2
EPS = 1e-16

NC = 2
NS = 16
NW = NC * NS
CHUNK = 128
NCH = 79
EPW = CHUNK * NCH
E_PAD = EPW * NW
NP = 10240
RPT = NP // NS
BIN = N

_f32 = jnp.float32


def _vmesh():
    return plsc.VectorSubcoreMesh(core_axis_name="c", subcore_axis_name="s",
                                  num_cores=NC)


def _zero_shared(zblk, u_sh, sid, width):
    z = jnp.zeros((16,), _f32)
    for i in range(16):
        for j in range(width // 16):
            zblk[i, pl.ds(j * 16, 16)] = z

    @pl.loop(0, RPT // 16)
    def _(r):
        pltpu.sync_copy(zblk, u_sh.at[pl.ds(sid * RPT + r * 16, 16)])


@functools.partial(
    pl.kernel,
    out_type=(
        jax.ShapeDtypeStruct((E_PAD,), _f32),
        jax.ShapeDtypeStruct((NC, NP, HID), _f32),
        jax.ShapeDtypeStruct((NC, NP), _f32),
    ),
    mesh=_vmesh(),
    scratch_types=[
        pltpu.VMEM((1, CHUNK), jnp.int32),
        pltpu.VMEM((1, CHUNK), jnp.int32),
        pltpu.VMEM((CHUNK, HID), _f32),
        pltpu.VMEM((CHUNK, HID), _f32),
        pltpu.VMEM((CHUNK,), _f32),
        pltpu.VMEM((HID,), _f32),
        pltpu.VMEM((16, HID), _f32),
        pltpu.VMEM_SHARED((NP, HID), _f32),
        pltpu.VMEM_SHARED((NP,), _f32),
        pltpu.SemaphoreType.DMA,
        pltpu.SemaphoreType.DMA,
    ],
)
def _e1(xl_hbm, att_hbm, src_hbm, dst_hbm, ex_hbm, u1_hbm, den_hbm,
        idx_s, idx_d, rows_s, rows_d, exc, attv, zblk, u_sh, den_sh,
        sem1, sem2):
    cid = lax.axis_index("c")
    sid = lax.axis_index("s")
    wid = cid * NS + sid

    _zero_shared(zblk, u_sh, sid, HID)

    @pl.loop(0, RPT // 16)
    def _(r):
        pltpu.sync_copy(zblk.at[0, pl.ds(0, 16)],
                        den_sh.at[pl.ds(sid * RPT + r * 16, 16)])

    pltpu.sync_copy(att_hbm, attv)
    av = [attv[pl.ds(16 * k, 16)] for k in range(HID // 16)]
    plsc.subcore_barrier()

    @pl.loop(0, NCH)
    def _(t):
        base = wid * EPW + t * CHUNK
        pltpu.sync_copy(src_hbm.at[pl.ds(base, CHUNK)], idx_s.at[0])
        pltpu.sync_copy(dst_hbm.at[pl.ds(base, CHUNK)], idx_d.at[0])
        cs = pltpu.async_copy(xl_hbm.at[idx_s.at[0]], rows_s, sem1)
        cd = pltpu.async_copy(xl_hbm.at[idx_d.at[0]], rows_d, sem2)
        cs.wait()
        cd.wait()

        @pl.loop(0, CHUNK)
        def _(e):
            acc = None
            for k in range(HID // 16):
                sl = pl.ds(16 * k, 16)
                f = rows_s[e, sl] + rows_d[e, sl]
                f = jnp.maximum(f, NEG * f)
                term = av[k] * f
                acc = term if acc is None else acc + term
            exc[e] = jnp.sum(acc)

        @pl.loop(0, CHUNK // 16)
        def _(g):
            sl = pl.ds(16 * g, 16)
            exc[sl] = jnp.exp(exc[sl])

        @pl.loop(0, CHUNK)
        def _(e):
            xv = exc[e]
            for k in range(HID // 16):
                sl = pl.ds(16 * k, 16)
                rows_s[e, sl] = rows_s[e, sl] * xv

        pltpu.sync_copy(exc, ex_hbm.at[pl.ds(base, CHUNK)])
        pltpu.sync_copy(rows_s, u_sh.at[idx_d.at[0]], add=True)
        pltpu.sync_copy(exc, den_sh.at[idx_d.at[0]], add=True)

    plsc.subcore_barrier()
    sl_out = pl.ds(sid * RPT, RPT)
    pltpu.sync_copy(u_sh.at[sl_out], u1_hbm.at[cid, sl_out])
    pltpu.sync_copy(den_sh.at[sl_out], den_hbm.at[cid, sl_out])


def _make_agg(width, with_ex):
    scratch = [
        pltpu.VMEM((1, CHUNK), jnp.int32),
        pltpu.VMEM((1, CHUNK), jnp.int32),
        pltpu.VMEM((CHUNK, width), _f32),
        pltpu.VMEM((16, width), _f32),
        pltpu.VMEM_SHARED((NP, width), _f32),
        pltpu.SemaphoreType.DMA,
    ]
    if with_ex:
        scratch.insert(3, pltpu.VMEM((CHUNK,), _f32))

    @functools.partial(
        pl.kernel,
        out_type=jax.ShapeDtypeStruct((NC, NP, width), _f32),
        mesh=_vmesh(),
        scratch_types=scratch,
    )
    def agg(*args):
        if with_ex:
            (tab_hbm, src_hbm, dst_hbm, ex_hbm, u_hbm,
             idx_s, idx_d, rows, exc, zblk, u_sh, sem) = args
        else:
            (tab_hbm, src_hbm, dst_hbm, u_hbm,
             idx_s, idx_d, rows, zblk, u_sh, sem) = args
        cid = lax.axis_index("c")
        sid = lax.axis_index("s")
        wid = cid * NS + sid

        _zero_shared(zblk, u_sh, sid, width)
        plsc.subcore_barrier()

        @pl.loop(0, NCH)
        def _(t):
            base = wid * EPW + t * CHUNK
            pltpu.sync_copy(src_hbm.at[pl.ds(base, CHUNK)], idx_s.at[0])
            pltpu.sync_copy(dst_hbm.at[pl.ds(base, CHUNK)], idx_d.at[0])
            pltpu.async_copy(tab_hbm.at[idx_s.at[0]], rows, sem).wait()
            if with_ex:
                pltpu.sync_copy(ex_hbm.at[pl.ds(base, CHUNK)], exc)

                @pl.loop(0, CHUNK)
                def _(e):
                    xv = exc[e]
                    for k in range(width // 16):
                        sl = pl.ds(16 * k, 16)
                        rows[e, sl] = rows[e, sl] * xv

            pltpu.sync_copy(rows, u_sh.at[idx_d.at[0]], add=True)

        plsc.subcore_barrier()
        sl_out = pl.ds(sid * RPT, RPT)
        pltpu.sync_copy(u_sh.at[sl_out], u_hbm.at[cid, sl_out])

    return agg


_agg32 = _make_agg(OUT_DIM, False)
_agg32ex = _make_agg(OUT_DIM, True)
_agg64 = _make_agg(HID, False)


def _mm_body(x_ref, w_ref, o_ref):
    o_ref[...] = jnp.dot(x_ref[...], w_ref[...],
                         preferred_element_type=_f32)


def _k2_body(u_ref, d_ref, w_ref, o_ref):
    u = u_ref[0] + u_ref[1]
    d = d_ref[0] + d_ref[1]
    h1 = u[:N] / (d[:N] + EPS)
    h1 = jnp.where(h1 > 0, h1, jnp.exp(h1) - 1.0)
    o_ref[...] = jnp.dot(h1, w_ref[...], preferred_element_type=_f32)


def _k3_body(u_ref, o_ref):
    o_ref[...] = u_ref[0, :N] + u_ref[1, :N]


def _k4_body(u_ref, d_ref, w_ref, o_ref):
    u = u_ref[0] + u_ref[1]
    d = d_ref[0] + d_ref[1]
    m = u[:N] / (d[:N] + EPS)
    t = jnp.dot(m, w_ref[...], preferred_element_type=_f32)
    o_ref[...] = jnp.where(t > 0, t, jnp.exp(t) - 1.0)


def _k5_body(u_ref, w_ref, o_ref):
    o_ref[...] = jnp.dot(u_ref[0, :N] + u_ref[1, :N], w_ref[...],
                         preferred_element_type=_f32)


def _tc(body, out_shape, *args):
    return pl.pallas_call(
        body, out_shape=jax.ShapeDtypeStruct(out_shape, _f32))(*args)


def kernel(features, edge_index, W1, att1, W2, W4):
    src = edge_index[0]
    dst = edge_index[1]
    pad = E_PAD - src.shape[0]
    srcp = jnp.concatenate([src, jnp.zeros((pad,), jnp.int32)])
    dstp = jnp.concatenate([dst, jnp.full((pad,), BIN, jnp.int32)])
    w2t = W2.T

    xl = _tc(_mm_body, (N, HID), features, W1)
    ex, u1, den = _e1(xl, att1, srcp, dstp)
    den = den.reshape(NC, NP, 1)
    g2 = _tc(_k2_body, (N, OUT_DIM), u1, den, W2)
    u2 = _agg32(g2, srcp, dstp)
    h2 = _tc(_k3_body, (N, OUT_DIM), u2)
    u3 = _agg32ex(h2, srcp, dstp, ex)
    h3 = _tc(_k4_body, (N, HID), u3, den, w2t)
    u4 = _agg64(h3, srcp, dstp)
    h4 = _tc(_k5_body, (N, IN_DIM), u4, W4)
    return (h2, h4)


# speedup vs baseline: 7.2854x; 7.2854x over previous
import dataclasses
import functools

import jax
import jax.numpy as jnp
from jax import lax
from jax.experimental import pallas as pl
from jax.experimental.pallas import tpu as pltpu
from jax.experimental.pallas import tpu_sc as plsc

N = 10000
IN_DIM = 128
HID = 64
OUT_DIM = 32
NEG = 0.2
EPS = 1e-16

NC = 2
NS = 16
NW = NC * NS
CHUNK = 128
NCH = 79
EPW = CHUNK * NCH
E_PAD = EPW * NW
NP = 10240
RPT = NP // NS
BIN = N

_f32 = jnp.float32


def _vmesh():
    return plsc.VectorSubcoreMesh(core_axis_name="c", subcore_axis_name="s",
                                  num_cores=NC)


def _sc_params():
    cp = pltpu.CompilerParams()
    fields = pltpu.CompilerParams.__dataclass_fields__
    if "needs_layout_passes" in fields:
        cp = dataclasses.replace(cp, needs_layout_passes=False)
    if "use_tc_tiling_on_sc" in fields:
        cp = dataclasses.replace(cp, use_tc_tiling_on_sc=False)
    return cp


def _zero_shared(zblk, u_sh, sid, width):
    z = jnp.zeros((16,), _f32)
    for i in range(16):
        for j in range(width // 16):
            zblk[i, pl.ds(j * 16, 16)] = z

    @pl.loop(0, RPT // 16)
    def _(r):
        pltpu.sync_copy(zblk, u_sh.at[pl.ds(sid * RPT + r * 16, 16)])


@functools.partial(
    pl.kernel,
    out_type=(
        jax.ShapeDtypeStruct((E_PAD,), _f32),
        jax.ShapeDtypeStruct((NC, NP, HID), _f32),
        jax.ShapeDtypeStruct((NC, NP), _f32),
    ),
    mesh=_vmesh(),
    compiler_params=_sc_params(),
    scratch_types=[
        pltpu.VMEM((1, CHUNK), jnp.int32),
        pltpu.VMEM((1, CHUNK), jnp.int32),
        pltpu.VMEM((CHUNK, HID), _f32),
        pltpu.VMEM((CHUNK, HID), _f32),
        pltpu.VMEM((CHUNK,), _f32),
        pltpu.VMEM((HID,), _f32),
        pltpu.VMEM((16, HID), _f32),
        pltpu.VMEM_SHARED((NP, HID), _f32),
        pltpu.VMEM_SHARED((NP,), _f32),
        pltpu.SemaphoreType.DMA,
        pltpu.SemaphoreType.DMA,
    ],
)
def _e1(xl_hbm, att_hbm, src_hbm, dst_hbm, ex_hbm, u1_hbm, den_hbm,
        idx_s, idx_d, rows_s, rows_d, exc, attv, zblk, u_sh, den_sh,
        sem1, sem2):
    cid = lax.axis_index("c")
    sid = lax.axis_index("s")
    wid = cid * NS + sid

    _zero_shared(zblk, u_sh, sid, HID)

    @pl.loop(0, RPT // 16)
    def _(r):
        pltpu.sync_copy(zblk.at[0, pl.ds(0, 16)],
                        den_sh.at[pl.ds(sid * RPT + r * 16, 16)])

    pltpu.sync_copy(att_hbm, attv)
    av = [attv[pl.ds(16 * k, 16)] for k in range(HID // 16)]
    lane0 = lax.iota(jnp.int32, 16) == 0
    plsc.subcore_barrier()

    @pl.loop(0, NCH)
    def _(t):
        base = wid * EPW + t * CHUNK
        pltpu.sync_copy(src_hbm.at[pl.ds(base, CHUNK)], idx_s.at[0])
        pltpu.sync_copy(dst_hbm.at[pl.ds(base, CHUNK)], idx_d.at[0])
        cs = pltpu.async_copy(xl_hbm.at[idx_s.at[0]], rows_s, sem1)
        cd = pltpu.async_copy(xl_hbm.at[idx_d.at[0]], rows_d, sem2)
        cs.wait()
        cd.wait()

        @pl.loop(0, CHUNK)
        def _(e):
            acc = None
            for k in range(HID // 16):
                sl = pl.ds(16 * k, 16)
                f = rows_s[e, sl] + rows_d[e, sl]
                f = jnp.maximum(f, NEG * f)
                term = av[k] * f
                acc = term if acc is None else acc + term
            s = jnp.sum(acc)
            plsc.store_scatter(exc, [jnp.full((16,), e, jnp.int32)],
                               jnp.full((16,), s, _f32), mask=lane0)

        @pl.loop(0, CHUNK // 16)
        def _(g):
            sl = pl.ds(16 * g, 16)
            exc[sl] = jnp.exp(exc[sl])

        @pl.loop(0, CHUNK // 16)
        def _(g):
            ev = exc[pl.ds(16 * g, 16)]
            for j in range(16):
                xv = ev[j]
                e = 16 * g + j
                for k in range(HID // 16):
                    sl = pl.ds(16 * k, 16)
                    rows_s[e, sl] = rows_s[e, sl] * xv

        pltpu.sync_copy(exc, ex_hbm.at[pl.ds(base, CHUNK)])
        pltpu.sync_copy(rows_s, u_sh.at[idx_d.at[0]], add=True)
        pltpu.sync_copy(exc, den_sh.at[idx_d.at[0]], add=True)

    plsc.subcore_barrier()
    sl_out = pl.ds(sid * RPT, RPT)
    pltpu.sync_copy(u_sh.at[sl_out], u1_hbm.at[cid, sl_out])
    pltpu.sync_copy(den_sh.at[sl_out], den_hbm.at[cid, sl_out])


def _make_agg(width, with_ex):
    scratch = [
        pltpu.VMEM((1, CHUNK), jnp.int32),
        pltpu.VMEM((1, CHUNK), jnp.int32),
        pltpu.VMEM((CHUNK, width), _f32),
        pltpu.VMEM((16, width), _f32),
        pltpu.VMEM_SHARED((NP, width), _f32),
        pltpu.SemaphoreType.DMA,
    ]
    if with_ex:
        scratch.insert(3, pltpu.VMEM((CHUNK,), _f32))

    @functools.partial(
        pl.kernel,
        out_type=jax.ShapeDtypeStruct((NC, NP, width), _f32),
        mesh=_vmesh(),
        compiler_params=_sc_params(),
        scratch_types=scratch,
    )
    def agg(*args):
        if with_ex:
            (tab_hbm, src_hbm, dst_hbm, ex_hbm, u_hbm,
             idx_s, idx_d, rows, exc, zblk, u_sh, sem) = args
        else:
            (tab_hbm, src_hbm, dst_hbm, u_hbm,
             idx_s, idx_d, rows, zblk, u_sh, sem) = args
        cid = lax.axis_index("c")
        sid = lax.axis_index("s")
        wid = cid * NS + sid

        _zero_shared(zblk, u_sh, sid, width)
        plsc.subcore_barrier()

        @pl.loop(0, NCH)
        def _(t):
            base = wid * EPW + t * CHUNK
            pltpu.sync_copy(src_hbm.at[pl.ds(base, CHUNK)], idx_s.at[0])
            pltpu.sync_copy(dst_hbm.at[pl.ds(base, CHUNK)], idx_d.at[0])
            pltpu.async_copy(tab_hbm.at[idx_s.at[0]], rows, sem).wait()
            if with_ex:
                pltpu.sync_copy(ex_hbm.at[pl.ds(base, CHUNK)], exc)

                @pl.loop(0, CHUNK // 16)
                def _(g):
                    ev = exc[pl.ds(16 * g, 16)]
                    for j in range(16):
                        xv = ev[j]
                        e = 16 * g + j
                        for k in range(width // 16):
                            sl = pl.ds(16 * k, 16)
                            rows[e, sl] = rows[e, sl] * xv

            pltpu.sync_copy(rows, u_sh.at[idx_d.at[0]], add=True)

        plsc.subcore_barrier()
        sl_out = pl.ds(sid * RPT, RPT)
        pltpu.sync_copy(u_sh.at[sl_out], u_hbm.at[cid, sl_out])

    return agg


_agg32 = _make_agg(OUT_DIM, False)
_agg32ex = _make_agg(OUT_DIM, True)
_agg64 = _make_agg(HID, False)


def _mm_body(x_ref, w_ref, o_ref):
    o_ref[...] = jnp.dot(x_ref[...], w_ref[...],
                         preferred_element_type=_f32)


def _k2_body(u_ref, d_ref, w_ref, o_ref):
    u = u_ref[0] + u_ref[1]
    d = d_ref[0] + d_ref[1]
    h1 = u[:N] / (d[:N] + EPS)
    h1 = jnp.where(h1 > 0, h1, jnp.exp(h1) - 1.0)
    o_ref[...] = jnp.dot(h1, w_ref[...], preferred_element_type=_f32)


def _k3_body(u_ref, o_ref):
    o_ref[...] = u_ref[0, :N] + u_ref[1, :N]


def _k4_body(u_ref, d_ref, w_ref, o_ref):
    u = u_ref[0] + u_ref[1]
    d = d_ref[0] + d_ref[1]
    m = u[:N] / (d[:N] + EPS)
    t = jnp.dot(m, w_ref[...], preferred_element_type=_f32)
    o_ref[...] = jnp.where(t > 0, t, jnp.exp(t) - 1.0)


def _k5_body(u_ref, w_ref, o_ref):
    o_ref[...] = jnp.dot(u_ref[0, :N] + u_ref[1, :N], w_ref[...],
                         preferred_element_type=_f32)


def _tc(body, out_shape, *args):
    return pl.pallas_call(
        body, out_shape=jax.ShapeDtypeStruct(out_shape, _f32))(*args)


def kernel(features, edge_index, W1, att1, W2, W4):
    src = edge_index[0]
    dst = edge_index[1]
    pad = E_PAD - src.shape[0]
    srcp = jnp.concatenate([src, jnp.zeros((pad,), jnp.int32)])
    dstp = jnp.concatenate([dst, jnp.full((pad,), BIN, jnp.int32)])
    w2t = W2.T

    xl = _tc(_mm_body, (N, HID), features, W1)
    ex, u1, den = _e1(xl, att1, srcp, dstp)
    den = den.reshape(NC, NP, 1)
    g2 = _tc(_k2_body, (N, OUT_DIM), u1, den, W2)
    u2 = _agg32(g2, srcp, dstp)
    h2 = _tc(_k3_body, (N, OUT_DIM), u2)
    u3 = _agg32ex(h2, srcp, dstp, ex)
    h3 = _tc(_k4_body, (N, HID), u3, den, w2t)
    u4 = _agg64(h3, srcp, dstp)
    h4 = _tc(_k5_body, (N, IN_DIM), u4, W4)
    return (h2, h4)


# speedup vs baseline: 9.0411x; 1.2410x over previous
import dataclasses
import functools

import jax
import jax.numpy as jnp
from jax import lax
from jax.experimental import pallas as pl
from jax.experimental.pallas import tpu as pltpu
from jax.experimental.pallas import tpu_sc as plsc

N = 10000
IN_DIM = 128
HID = 64
OUT_DIM = 32
NEG = 0.2
EPS = 1e-16

NC = 2
NS = 16
NW = NC * NS
CHUNK = 128
NCH = 81
EPW = CHUNK * NCH
E_PAD = EPW * NW
NP = 10240
RPT = NP // NS
BIN = N

_f32 = jnp.float32


def _vmesh():
    return plsc.VectorSubcoreMesh(core_axis_name="c", subcore_axis_name="s",
                                  num_cores=NC)


def _sc_params():
    cp = pltpu.CompilerParams()
    fields = pltpu.CompilerParams.__dataclass_fields__
    if "needs_layout_passes" in fields:
        cp = dataclasses.replace(cp, needs_layout_passes=False)
    if "use_tc_tiling_on_sc" in fields:
        cp = dataclasses.replace(cp, use_tc_tiling_on_sc=False)
    return cp


def _zero_shared(zblk, u_sh, sid, width):
    z = jnp.zeros((16,), _f32)
    for i in range(16):
        for j in range(width // 16):
            zblk[i, pl.ds(j * 16, 16)] = z

    @pl.loop(0, RPT // 16)
    def _(r):
        pltpu.sync_copy(zblk, u_sh.at[pl.ds(sid * RPT + r * 16, 16)])


@functools.partial(
    pl.kernel,
    out_type=(
        jax.ShapeDtypeStruct((NW, NCH, CHUNK), _f32),
        jax.ShapeDtypeStruct((NC, NP, HID), _f32),
        jax.ShapeDtypeStruct((NC, NP), _f32),
    ),
    mesh=_vmesh(),
    compiler_params=_sc_params(),
    scratch_types=[
        pltpu.VMEM((NCH, CHUNK), jnp.int32),
        pltpu.VMEM((NCH, CHUNK), jnp.int32),
        pltpu.VMEM((3, CHUNK, HID), _f32),
        pltpu.VMEM((3, CHUNK, HID), _f32),
        pltpu.VMEM((NCH, CHUNK), _f32),
        pltpu.VMEM((HID,), _f32),
        pltpu.VMEM((16, HID), _f32),
        pltpu.VMEM_SHARED((NP, HID), _f32),
        pltpu.VMEM_SHARED((NP,), _f32),
        pltpu.SemaphoreType.DMA,
        pltpu.SemaphoreType.DMA,
        pltpu.SemaphoreType.DMA,
        pltpu.SemaphoreType.DMA,
        pltpu.SemaphoreType.DMA,
        pltpu.SemaphoreType.DMA,
        pltpu.SemaphoreType.DMA,
        pltpu.SemaphoreType.DMA,
        pltpu.SemaphoreType.DMA,
        pltpu.SemaphoreType.DMA,
    ],
)
def _e1(xl_hbm, att_hbm, src_hbm, dst_hbm, ex_hbm, u1_hbm, den_hbm,
        idxs, idxd, rs_ring, rd_ring, exc, attv, zblk, u_sh, den_sh,
        gs0, gs1, gs2, gd0, gd1, gd2, ss0, ss1, ss2, densem):
    cid = lax.axis_index("c")
    sid = lax.axis_index("s")
    wid = cid * NS + sid
    gs = (gs0, gs1, gs2)
    gd = (gd0, gd1, gd2)
    ss = (ss0, ss1, ss2)
    rs = [rs_ring.at[b] for b in range(3)]
    rd = [rd_ring.at[b] for b in range(3)]

    pltpu.sync_copy(src_hbm.at[wid], idxs)
    pltpu.sync_copy(dst_hbm.at[wid], idxd)
    pltpu.sync_copy(att_hbm, attv)

    def issue_gather(t, b):
        pltpu.async_copy(xl_hbm.at[idxs.at[t]], rs[b], gs[b])
        pltpu.async_copy(xl_hbm.at[idxd.at[t]], rd[b], gd[b])

    def wait_gather(b):
        pltpu.make_async_copy(xl_hbm.at[idxs.at[0]], rs[b], gs[b]).wait()
        pltpu.make_async_copy(xl_hbm.at[idxd.at[0]], rd[b], gd[b]).wait()

    def issue_scatter(t, b):
        pltpu.async_copy(rs[b], u_sh.at[idxd.at[t]], ss[b], add=True)

    def wait_scatter(b):
        pltpu.make_async_copy(rs[b], u_sh.at[idxd.at[0]], ss[b]).wait()

    issue_gather(0, 0)
    issue_gather(1, 1)

    _zero_shared(zblk, u_sh, sid, HID)

    @pl.loop(0, RPT // 16)
    def _(r):
        pltpu.sync_copy(zblk.at[0, pl.ds(0, 16)],
                        den_sh.at[pl.ds(sid * RPT + r * 16, 16)])

    av = [attv[pl.ds(16 * k, 16)] for k in range(HID // 16)]
    lane15 = lax.iota(jnp.int32, 16) == 15
    plsc.subcore_barrier()

    @pl.loop(0, NCH // 3)
    def _(i):
        for db in range(3):
            t = 3 * i + db
            wait_gather(db)
            rsb, rdb = rs[db], rd[db]
            exrow = exc.at[t]

            @pl.loop(0, CHUNK // 16)
            def _(g):
                for j in range(16):
                    e_idx = 16 * g + j
                    a = []
                    acc = None
                    for k in range(HID // 16):
                        sl = pl.ds(16 * k, 16)
                        ak = rsb[e_idx, sl]
                        a.append(ak)
                        f = ak + rdb[e_idx, sl]
                        f = jnp.maximum(f, NEG * f)
                        term = av[k] * f
                        acc = term if acc is None else acc + term
                    exv = jnp.exp(plsc.cumsum(acc))
                    plsc.store_scatter(
                        exrow, [jnp.full((16,), e_idx, jnp.int32)], exv,
                        mask=lane15)
                    xs = exv[15]
                    for k in range(HID // 16):
                        sl = pl.ds(16 * k, 16)
                        rsb[e_idx, sl] = a[k] * xs

            issue_scatter(t, db)
            nb = (db + 2) % 3

            @pl.when(t + 2 < NCH)
            def _():
                @pl.when(t >= 1)
                def _():
                    wait_scatter(nb)

                issue_gather(t + 2, nb)

    for b in range(3):
        wait_scatter(b)

    @pl.loop(0, NCH // 9)
    def _(q):
        for r in range(9):
            t = 9 * q + r
            pltpu.async_copy(exc.at[t], den_sh.at[idxd.at[t]], densem,
                             add=True)
        for r in range(9):
            pltpu.make_async_copy(exc.at[0], den_sh.at[idxd.at[0]],
                                  densem).wait()

    pltpu.sync_copy(exc, ex_hbm.at[wid])
    plsc.subcore_barrier()
    sl_out = pl.ds(sid * RPT, RPT)
    pltpu.sync_copy(u_sh.at[sl_out], u1_hbm.at[cid, sl_out])
    pltpu.sync_copy(den_sh.at[sl_out], den_hbm.at[cid, sl_out])


def _make_agg(width, with_ex):
    scratch = [
        pltpu.VMEM((NCH, CHUNK), jnp.int32),
        pltpu.VMEM((NCH, CHUNK), jnp.int32),
        pltpu.VMEM((3, CHUNK, width), _f32),
        pltpu.VMEM((16, width), _f32),
        pltpu.VMEM_SHARED((NP, width), _f32),
        pltpu.SemaphoreType.DMA,
        pltpu.SemaphoreType.DMA,
        pltpu.SemaphoreType.DMA,
        pltpu.SemaphoreType.DMA,
        pltpu.SemaphoreType.DMA,
        pltpu.SemaphoreType.DMA,
    ]
    if with_ex:
        scratch.insert(3, pltpu.VMEM((NCH, CHUNK), _f32))

    @functools.partial(
        pl.kernel,
        out_type=jax.ShapeDtypeStruct((NC, NP, width), _f32),
        mesh=_vmesh(),
        compiler_params=_sc_params(),
        scratch_types=scratch,
    )
    def agg(*args):
        if with_ex:
            (tab_hbm, src_hbm, dst_hbm, ex_hbm, u_hbm,
             idxs, idxd, ring, exc, zblk, u_sh,
             g0, g1, g2, s0, s1, s2) = args
        else:
            (tab_hbm, src_hbm, dst_hbm, u_hbm,
             idxs, idxd, ring, zblk, u_sh,
             g0, g1, g2, s0, s1, s2) = args
        cid = lax.axis_index("c")
        sid = lax.axis_index("s")
        wid = cid * NS + sid
        gsem = (g0, g1, g2)
        ssem = (s0, s1, s2)
        rows = [ring.at[b] for b in range(3)]

        pltpu.sync_copy(src_hbm.at[wid], idxs)
        pltpu.sync_copy(dst_hbm.at[wid], idxd)
        if with_ex:
            pltpu.sync_copy(ex_hbm.at[wid], exc)

        def issue_gather(t, b):
            pltpu.async_copy(tab_hbm.at[idxs.at[t]], rows[b], gsem[b])

        def wait_gather(b):
            pltpu.make_async_copy(tab_hbm.at[idxs.at[0]], rows[b],
                                  gsem[b]).wait()

        def issue_scatter(t, b):
            pltpu.async_copy(rows[b], u_sh.at[idxd.at[t]], ssem[b], add=True)

        def wait_scatter(b):
            pltpu.make_async_copy(rows[b], u_sh.at[idxd.at[0]],
                                  ssem[b]).wait()

        issue_gather(0, 0)
        issue_gather(1, 1)
        _zero_shared(zblk, u_sh, sid, width)
        plsc.subcore_barrier()

        @pl.loop(0, NCH // 3)
        def _(i):
            for db in range(3):
                t = 3 * i + db
                wait_gather(db)
                if with_ex:
                    rb = rows[db]

                    @pl.loop(0, CHUNK // 16)
                    def _(g):
                        ev = exc[t, pl.ds(16 * g, 16)]
                        for j in range(16):
                            xv = ev[j]
                            e_idx = 16 * g + j
                            for k in range(width // 16):
                                sl = pl.ds(16 * k, 16)
                                rb[e_idx, sl] = rb[e_idx, sl] * xv

                issue_scatter(t, db)
                nb = (db + 2) % 3

                @pl.when(t + 2 < NCH)
                def _():
                    @pl.when(t >= 1)
                    def _():
                        wait_scatter(nb)

                    issue_gather(t + 2, nb)

        for b in range(3):
            wait_scatter(b)

        plsc.subcore_barrier()
        sl_out = pl.ds(sid * RPT, RPT)
        pltpu.sync_copy(u_sh.at[sl_out], u_hbm.at[cid, sl_out])

    return agg


_agg32 = _make_agg(OUT_DIM, False)
_agg32ex = _make_agg(OUT_DIM, True)
_agg64 = _make_agg(HID, False)


def _mm_body(x_ref, w_ref, o_ref):
    o_ref[...] = jnp.dot(x_ref[...], w_ref[...],
                         preferred_element_type=_f32)


def _k2_body(u_ref, d_ref, w_ref, o_ref):
    u = u_ref[0] + u_ref[1]
    d = d_ref[0] + d_ref[1]
    h1 = u[:N] / (d[:N] + EPS)
    h1 = jnp.where(h1 > 0, h1, jnp.exp(h1) - 1.0)
    o_ref[...] = jnp.dot(h1, w_ref[...], preferred_element_type=_f32)


def _k3_body(u_ref, o_ref):
    o_ref[...] = u_ref[0, :N] + u_ref[1, :N]


def _k4_body(u_ref, d_ref, w_ref, o_ref):
    u = u_ref[0] + u_ref[1]
    d = d_ref[0] + d_ref[1]
    m = u[:N] / (d[:N] + EPS)
    t = jnp.dot(m, w_ref[...], preferred_element_type=_f32)
    o_ref[...] = jnp.where(t > 0, t, jnp.exp(t) - 1.0)


def _k5_body(u_ref, w_ref, o_ref):
    o_ref[...] = jnp.dot(u_ref[0, :N] + u_ref[1, :N], w_ref[...],
                         preferred_element_type=_f32)


def _tc(body, out_shape, *args):
    return pl.pallas_call(
        body, out_shape=jax.ShapeDtypeStruct(out_shape, _f32))(*args)


def kernel(features, edge_index, W1, att1, W2, W4):
    src = edge_index[0]
    dst = edge_index[1]
    pad = E_PAD - src.shape[0]
    srcp = jnp.concatenate([src, jnp.zeros((pad,), jnp.int32)])
    dstp = jnp.concatenate([dst, jnp.full((pad,), BIN, jnp.int32)])
    srcp = srcp.reshape(NW, NCH, CHUNK)
    dstp = dstp.reshape(NW, NCH, CHUNK)
    w2t = W2.T

    xl = _tc(_mm_body, (N, HID), features, W1)
    ex, u1, den = _e1(xl, att1, srcp, dstp)
    den = den.reshape(NC, NP, 1)
    g2 = _tc(_k2_body, (N, OUT_DIM), u1, den, W2)
    u2 = _agg32(g2, srcp, dstp)
    h2 = _tc(_k3_body, (N, OUT_DIM), u2)
    u3 = _agg32ex(h2, srcp, dstp, ex)
    h3 = _tc(_k4_body, (N, HID), u3, den, w2t)
    u4 = _agg64(h3, srcp, dstp)
    h4 = _tc(_k5_body, (N, IN_DIM), u4, W4)
    return (h2, h4)
